# baseline (speedup 1.0000x reference)
import functools

import jax
import jax.numpy as jnp
from jax import lax
from jax.experimental import pallas as pl
from jax.experimental.pallas import tpu as pltpu

_J = 8


def _shift_lanes(v, off):
    n = v.shape[-1]
    k = off % n
    if k == 0:
        return v
    return jnp.concatenate([v[..., k:], v[..., :k]], axis=-1)


def _conv_lin1_kernel(x_ref, w8_ref, w1_ref, b1_ref, rep_ref, h_ref,
                      *, H, W, C, F, KH, KW):
    HW = H * W
    Bblk = x_ref.shape[3]
    Gblk = Bblk // _J
    Hd = w1_ref.shape[1]

    xm = x_ref[...].reshape(C, HW, Bblk)
    xt = jnp.transpose(xm, (0, 2, 1))
    xq = jnp.transpose(xt.reshape(C, Gblk, _J, HW), (1, 0, 2, 3))
    xb = xq.reshape(Gblk, C * _J, HW).astype(jnp.bfloat16)

    lane = lax.broadcasted_iota(jnp.int32, (1, 1, HW), 2)
    yy = lane // W
    xx = lane - yy * W

    tiles = []
    for oy in range(-(KH // 2), KH - KH // 2):
        for ox in range(-(KW // 2), KW - KW // 2):
            m = ((yy + oy >= 0) & (yy + oy < H) &
                 (xx + ox >= 0) & (xx + ox < W))
            tiles.append(jnp.where(m, _shift_lanes(xb, oy * W + ox),
                                   jnp.bfloat16(0)))
    tiles.append(jnp.ones((Gblk, _J, HW), jnp.bfloat16))
    patch = jnp.concatenate(tiles, axis=1)

    w8 = w8_ref[...]
    acts = []
    for g in range(Gblk):
        cg = jnp.dot(w8, patch[g],
                     preferred_element_type=jnp.float32)
        acts.append(jnp.maximum(cg, 0.0))
    conv = jnp.stack(acts, axis=0)

    repb = conv.reshape(Bblk, F * HW)
    rep_ref[...] = repb

    h = jnp.zeros((Bblk, Hd), jnp.float32)
    for f in range(F):
        h = h + jnp.dot(repb[:, f * HW:(f + 1) * HW],
                        w1_ref[f * HW:(f + 1) * HW, :],
                        preferred_element_type=jnp.float32)
    h_ref[...] = h + b1_ref[...]


def _bn_lin2_kernel(h_ref, g_ref, bt_ref, w2_ref, b2_ref, out_ref):
    h = h_ref[...]
    B = h.shape[0]
    s1 = jnp.sum(h, axis=0, keepdims=True)
    s2 = jnp.sum(h * h, axis=0, keepdims=True)
    mean = s1 * (1.0 / B)
    var = s2 * (1.0 / B) - mean * mean
    scale = g_ref[...] * lax.rsqrt(var + 1e-5)
    shift = bt_ref[...] - mean * scale
    hn = jnp.maximum(h * scale + shift, 0.0)
    out_ref[...] = (jnp.dot(hn, w2_ref[...],
                            preferred_element_type=jnp.float32) + b2_ref[...])


def kernel(x, conv_w, conv_b, w1, b1, gamma, beta, w2, b2):
    B, C, H, W = x.shape
    F, _, KH, KW = conv_w.shape
    HW = H * W
    D, Hd = w1.shape
    P = w2.shape[1]
    J = _J

    Bblk = min(256, B)
    while B % Bblk or Bblk % J:
        Bblk -= 1

    wtc = jnp.transpose(conv_w, (0, 2, 3, 1)).reshape(F, KH * KW, C)
    eyeJ = jnp.eye(J, dtype=conv_w.dtype)
    w8 = jnp.einsum('ij,ftc->iftcj', eyeJ, wtc)
    w8 = w8.reshape(J * F, KH * KW * J * C)
    bias_blk = jnp.pad(jnp.tile(conv_b, (J,)).reshape(J * F, 1),
                       ((0, 0), (0, J - 1)))
    w8 = jnp.concatenate([w8, bias_blk], axis=1).astype(jnp.bfloat16)
    b1r = b1.reshape(1, Hd)

    body = functools.partial(_conv_lin1_kernel, H=H, W=W, C=C, F=F,
                             KH=KH, KW=KW)
    rep, h = pl.pallas_call(
        body,
        out_shape=(jax.ShapeDtypeStruct((B, D), jnp.float32),
                   jax.ShapeDtypeStruct((B, Hd), jnp.float32)),
        grid=(B // Bblk,),
        in_specs=[
            pl.BlockSpec((C, H, W, Bblk), lambda i: (0, 0, 0, i)),
            pl.BlockSpec((J * F, KH * KW * J * C + J), lambda i: (0, 0)),
            pl.BlockSpec((D, Hd), lambda i: (0, 0)),
            pl.BlockSpec((1, Hd), lambda i: (0, 0)),
        ],
        out_specs=(
            pl.BlockSpec((Bblk, D), lambda i: (i, 0)),
            pl.BlockSpec((Bblk, Hd), lambda i: (i, 0)),
        ),
        compiler_params=pltpu.CompilerParams(
            dimension_semantics=("parallel",),
            vmem_limit_bytes=100 * 1024 * 1024,
        ),
    )(jnp.transpose(x, (1, 2, 3, 0)), w8, w1, b1r)

    def full(shape):
        return pl.BlockSpec(shape, lambda: (0,) * len(shape))

    projection = pl.pallas_call(
        _bn_lin2_kernel,
        out_shape=jax.ShapeDtypeStruct((B, P), jnp.float32),
        in_specs=[full((B, Hd)), full((1, Hd)), full((1, Hd)),
                  full((Hd, P)), full((1, P))],
        out_specs=full((B, P)),
    )(h, gamma.reshape(1, Hd), beta.reshape(1, Hd),
      w2, b2.reshape(1, P))

    return projection, rep
